# baseline (speedup 1.0000x reference)
import functools

import jax
import jax.numpy as jnp
from jax import lax
from jax.experimental import pallas as pl
from jax.experimental.pallas import tpu as pltpu
from jax.experimental.pallas import tpu_sc as plsc

_CHUNK = 128
_GRP = 4
_GROWS = _GRP * _CHUNK


def _build_gather(n_workers, n_chunks, d_model):
    mesh = plsc.VectorSubcoreMesh(core_axis_name="c", subcore_axis_name="s")
    n_rows = n_chunks * _CHUNK
    n_groups = n_chunks // _GRP
    num_cores = 2

    @functools.partial(
        pl.kernel,
        mesh=mesh,
        compiler_params=pltpu.CompilerParams(use_tc_tiling_on_sc=False),
        out_type=jax.ShapeDtypeStruct((n_workers * n_rows, d_model), jnp.float32),
        scratch_types=[
            pltpu.VMEM((n_chunks, _CHUNK), jnp.int32),
            pltpu.VMEM((2, _GROWS, d_model), jnp.float32),
            pltpu.SemaphoreType.DMA((2,)),
            pltpu.SemaphoreType.DMA((2,)),
        ],
    )
    def gather_kernel(table_hbm, idx_hbm, out_hbm, idx_v, rows_v, gsem, ssem):
        wid = lax.axis_index("s") * num_cores + lax.axis_index("c")
        base = wid * n_rows
        pltpu.sync_copy(idx_hbm.at[wid], idx_v)

        def issue_gathers(g, par):
            for q in range(_GRP):
                pltpu.async_copy(
                    table_hbm.at[idx_v.at[g * _GRP + q]],
                    rows_v.at[par, pl.ds(q * _CHUNK, _CHUNK)],
                    gsem.at[par],
                )

        def wait_gathers(par):
            pltpu.make_async_copy(
                table_hbm.at[pl.ds(0, _GROWS)], rows_v.at[par], gsem.at[par]
            ).wait()

        def start_store(g, par):
            pltpu.async_copy(
                rows_v.at[par],
                out_hbm.at[pl.ds(base + g * _GROWS, _GROWS)],
                ssem.at[par],
            )

        def wait_store(par):
            pltpu.make_async_copy(
                rows_v.at[par], out_hbm.at[pl.ds(base, _GROWS)], ssem.at[par]
            ).wait()

        issue_gathers(0, 0)
        issue_gathers(1, 1)
        wait_gathers(0)
        start_store(0, 0)

        def body(k, carry):
            g = 2 * k + 1
            wait_store(0)
            issue_gathers(g + 1, 0)
            wait_gathers(1)
            start_store(g, 1)
            wait_store(1)
            issue_gathers(g + 2, 1)
            wait_gathers(0)
            start_store(g + 1, 0)
            return carry

        lax.fori_loop(0, (n_groups - 2) // 2, body, 0)

        wait_gathers(1)
        start_store(n_groups - 1, 1)
        wait_store(0)
        wait_store(1)

    return gather_kernel


def kernel(token_ids, weight):
    b, s = token_ids.shape
    d_model = weight.shape[1]
    total = b * s
    flat = token_ids.reshape(-1).astype(jnp.int32)

    n_workers = 32
    grain = n_workers * _GROWS * 2
    padded = -(-total // grain) * grain
    if padded != total:
        flat = jnp.concatenate(
            [flat, jnp.zeros((padded - total,), jnp.int32)], axis=0
        )
    per_worker = padded // n_workers
    idx3 = flat.reshape(n_workers, per_worker // _CHUNK, _CHUNK)

    gather = _build_gather(n_workers, per_worker // _CHUNK, d_model)
    out = gather(weight, idx3)
    return out[:total].reshape(b, s, d_model)
